# baseline (speedup 1.0000x reference)
import functools

import jax
import jax.numpy as jnp
from jax import lax
from jax.experimental import pallas as pl
from jax.experimental.pallas import tpu as pltpu
from jax.experimental.pallas import tpu_sc as plsc

_PTS = 64
_IMP = 64
_EPS = 1e-5
_CH = 64
_L = 16


def _sc_call(n, op_f, dep_f, val_f, ori_f, dir_f):
    info = plsc.get_sparse_core_info()
    nc, ns = info.num_cores, info.num_subcores
    nw = nc * ns
    rpw = n // nw
    nch = rpw // _CH
    groups = _CH // _L

    mesh = plsc.VectorSubcoreMesh(core_axis_name="c", subcore_axis_name="s")

    @functools.partial(
        pl.kernel,
        out_type=jax.ShapeDtypeStruct((n * 198,), jnp.float32),
        mesh=mesh,
        scratch_types=[
            pltpu.VMEM((_CH * _PTS,), jnp.float32),
            pltpu.VMEM((_CH * _PTS,), jnp.float32),
            pltpu.VMEM((_CH * _PTS * 3,), jnp.float32),
            pltpu.VMEM((_CH * 3,), jnp.float32),
            pltpu.VMEM((_CH * 3,), jnp.float32),
            pltpu.VMEM((_CH * 198,), jnp.float32),
            pltpu.VMEM((_PTS * _L,), jnp.float32),
            pltpu.VMEM(((_IMP + 2) * _L,), jnp.int32),
        ],
        compiler_params=pltpu.CompilerParams(needs_layout_passes=False),
    )
    def body(op_h, dep_h, val_h, ori_h, dir_h, out_h,
             op_vm, dep_vm, val_vm, ori_vm, dir_vm, out_vm, cdf_vm, hist_vm):
        wid = lax.axis_index("s") * nc + lax.axis_index("c")
        iota = lax.iota(jnp.int32, _L)
        zf = jnp.zeros((_L,), jnp.float32)
        onef = jnp.ones((_L,), jnp.float32)
        onei = jnp.ones((_L,), jnp.int32)
        zi = jnp.zeros((_L,), jnp.int32)

        def group_body(g, _):
            ray = g * _L + iota
            ray64 = ray * _PTS
            ray192 = ray * (_PTS * 3)
            ray198 = ray * 198
            ray3 = ray * 3
            o0 = plsc.load_gather(ori_vm, [ray3])
            o1 = plsc.load_gather(ori_vm, [ray3 + 1])
            o2 = plsc.load_gather(ori_vm, [ray3 + 2])
            e0 = plsc.load_gather(dir_vm, [ray3])
            e1 = plsc.load_gather(dir_vm, [ray3 + 1])
            e2 = plsc.load_gather(dir_vm, [ray3 + 2])

            def march(k, carry):
                trans, cdf, a0, a1, a2 = carry
                opk = plsc.load_gather(op_vm, [ray64 + k])
                w = opk * trans
                trans = trans * (1.0 - opk)
                cdf = cdf + (w + _EPS)
                cdf_vm[pl.ds(k * _L, _L)] = cdf
                vb = ray192 + k * 3
                a0 = a0 + w * plsc.load_gather(val_vm, [vb])
                a1 = a1 + w * plsc.load_gather(val_vm, [vb + 1])
                a2 = a2 + w * plsc.load_gather(val_vm, [vb + 2])
                return trans, cdf, a0, a1, a2

            _, ctot, a0, a1, a2 = lax.fori_loop(
                0, _PTS, march, (onef, zf, zf, zf, zf), unroll=8)
            acc_o = jnp.clip(ctot - _PTS * _EPS, 0.0, 1.0)
            plsc.store_scatter(out_vm, [ray198], a0)
            plsc.store_scatter(out_vm, [ray198 + 1], a1)
            plsc.store_scatter(out_vm, [ray198 + 2], a2)
            plsc.store_scatter(out_vm, [ray198 + 3], acc_o)
            plsc.store_scatter(out_vm, [ray198 + 4], acc_o)
            plsc.store_scatter(out_vm, [ray198 + 5], acc_o)

            def hzero(v, _):
                hist_vm[pl.ds(v * _L, _L)] = zi
                return 0

            lax.fori_loop(0, _IMP + 2, hzero, 0, unroll=8)
            scale = jnp.float32(_IMP) / ctot

            def bink(k, _):
                ck = cdf_vm[pl.ds(k * _L, _L)]
                x = ck * scale
                xi = x.astype(jnp.int32)
                xi = xi + (xi.astype(jnp.float32) < x).astype(jnp.int32)
                m = jnp.minimum(xi, _IMP + 1)
                plsc.addupdate_scatter(hist_vm, [m * _L + iota], onei)
                return 0

            lax.fori_loop(0, _PTS, bink, 0, unroll=8)

            c_first = cdf_vm[pl.ds(0, _L)]
            d_first = plsc.load_gather(dep_vm, [ray64])
            d_last = plsc.load_gather(dep_vm, [ray64 + (_PTS - 1)])

            def sample(j, carry):
                f_prev, cnt = carry
                cnt = cnt + hist_vm[pl.ds(j * _L, _L)]
                i = jnp.clip(cnt, 1, _PTS - 1)
                g0 = (i - 1) * _L + iota
                c0 = plsc.load_gather(cdf_vm, [g0])
                c1 = plsc.load_gather(cdf_vm, [g0 + _L])
                di = ray64 + (i - 1)
                d0 = plsc.load_gather(dep_vm, [di])
                d1 = plsc.load_gather(dep_vm, [di + 1])
                uj = lax.convert_element_type(j, jnp.float32) * (1.0 / _IMP)
                u = uj * ctot
                f = d0 + ((u - c0) / (c1 - c0)) * (d1 - d0)
                f = jnp.where(u < c_first, d_first, f)
                f = jnp.where(u >= ctot, d_last, f)
                mid = 0.5 * (f_prev + f)
                ob = ray198 + (6 + (j - 1) * 3)
                plsc.store_scatter(out_vm, [ob], o0 + mid * e0)
                plsc.store_scatter(out_vm, [ob + 1], o1 + mid * e1)
                plsc.store_scatter(out_vm, [ob + 2], o2 + mid * e2)
                return f, cnt

            cnt0 = hist_vm[pl.ds(0, _L)]
            lax.fori_loop(1, _IMP + 1, sample, (d_first, cnt0), unroll=4)
            return 0

        def chunk_body(ch, _):
            base = wid * rpw + ch * _CH
            pltpu.sync_copy(op_h.at[pl.ds(base * _PTS, _CH * _PTS)], op_vm)
            pltpu.sync_copy(dep_h.at[pl.ds(base * _PTS, _CH * _PTS)], dep_vm)
            pltpu.sync_copy(val_h.at[pl.ds(base * _PTS * 3, _CH * _PTS * 3)],
                            val_vm)
            pltpu.sync_copy(ori_h.at[pl.ds(base * 3, _CH * 3)], ori_vm)
            pltpu.sync_copy(dir_h.at[pl.ds(base * 3, _CH * 3)], dir_vm)
            lax.fori_loop(0, groups, group_body, 0)
            pltpu.sync_copy(out_vm, out_h.at[pl.ds(base * 198, _CH * 198)])
            return 0

        lax.fori_loop(0, nch, chunk_body, 0)

    return body(op_f, dep_f, val_f, ori_f, dir_f)


def kernel(opacities, values, depths, origins, dirs):
    n = opacities.shape[0]
    out = _sc_call(
        n,
        opacities.reshape(-1),
        depths.reshape(-1),
        values.reshape(-1),
        origins.reshape(-1),
        dirs.reshape(-1),
    )
    return out.reshape(n, _IMP + 2, 3)


# speedup vs baseline: 6.0659x; 6.0659x over previous
import functools

import jax
import jax.numpy as jnp
from jax import lax
from jax.experimental import pallas as pl
from jax.experimental.pallas import tpu as pltpu
from jax.experimental.pallas import tpu_sc as plsc

_PTS = 64
_IMP = 64
_EPS = 1e-5
_CH = 64
_L = 16


def _sc_call(n, a_f, b_f, ori_f, dir_f):
    info = plsc.get_sparse_core_info()
    nc, ns = info.num_cores, info.num_subcores
    nw = nc * ns
    rpw = n // nw
    nch = rpw // _CH
    groups = _CH // _L

    mesh = plsc.VectorSubcoreMesh(core_axis_name="c", subcore_axis_name="s")

    @functools.partial(
        pl.kernel,
        out_type=jax.ShapeDtypeStruct((n * 256,), jnp.float32),
        mesh=mesh,
        scratch_types=[
            pltpu.VMEM((_CH * 128,), jnp.float32),
            pltpu.VMEM((_CH * 256,), jnp.float32),
            pltpu.VMEM((_CH * 3,), jnp.float32),
            pltpu.VMEM((_CH * 3,), jnp.float32),
            pltpu.VMEM((_CH * 256,), jnp.float32),
            pltpu.VMEM((_PTS * _L,), jnp.float32),
            pltpu.VMEM(((_IMP + 2) * _L,), jnp.int32),
        ],
        compiler_params=pltpu.CompilerParams(needs_layout_passes=False),
    )
    def body(a_h, b_h, ori_h, dir_h, out_h,
             a_vm, b_vm, ori_vm, dir_vm, out_vm, cdf_vm, hist_vm):
        wid = lax.axis_index("s") * nc + lax.axis_index("c")
        iota = lax.iota(jnp.int32, _L)
        zf = jnp.zeros((_L,), jnp.float32)
        onef = jnp.ones((_L,), jnp.float32)
        onei = jnp.ones((_L,), jnp.int32)
        zi = jnp.zeros((_L,), jnp.int32)

        def group_body(g, _):
            ray = g * _L + iota
            ray_a = ray * 128
            ray_b = ray * 256
            ray_y = ray * 256
            ray3 = ray * 3
            o0 = plsc.load_gather(ori_vm, [ray3])
            o1 = plsc.load_gather(ori_vm, [ray3 + 1])
            o2 = plsc.load_gather(ori_vm, [ray3 + 2])
            e0 = plsc.load_gather(dir_vm, [ray3])
            e1 = plsc.load_gather(dir_vm, [ray3 + 1])
            e2 = plsc.load_gather(dir_vm, [ray3 + 2])

            def march(k, carry):
                trans, cdf, a0, a1, a2 = carry
                opk = plsc.load_gather(a_vm, [ray_a + k])
                w = opk * trans
                trans = trans * (1.0 - opk)
                cdf = cdf + (w + _EPS)
                cdf_vm[pl.ds(k * _L, _L)] = cdf
                vb = ray_b + k
                a0 = a0 + w * plsc.load_gather(b_vm, [vb])
                a1 = a1 + w * plsc.load_gather(b_vm, [vb + 64])
                a2 = a2 + w * plsc.load_gather(b_vm, [vb + 128])
                return trans, cdf, a0, a1, a2

            _, ctot, a0, a1, a2 = lax.fori_loop(
                0, _PTS, march, (onef, zf, zf, zf, zf), unroll=8)
            acc_o = jnp.clip(ctot - _PTS * _EPS, 0.0, 1.0)
            plsc.store_scatter(out_vm, [ray_y], a0)
            plsc.store_scatter(out_vm, [ray_y + 1], a1)
            plsc.store_scatter(out_vm, [ray_y + 2], a2)
            plsc.store_scatter(out_vm, [ray_y + 3], acc_o)

            def hzero(v, _):
                hist_vm[pl.ds(v * _L, _L)] = zi
                return 0

            lax.fori_loop(0, _IMP + 2, hzero, 0, unroll=8)
            scale = jnp.float32(_IMP) / ctot

            def bink(k, _):
                ck = cdf_vm[pl.ds(k * _L, _L)]
                x = ck * scale
                xi = x.astype(jnp.int32)
                xi = xi + (xi.astype(jnp.float32) < x).astype(jnp.int32)
                m = jnp.minimum(xi, _IMP + 1)
                plsc.addupdate_scatter(hist_vm, [m * _L + iota], onei)
                return 0

            lax.fori_loop(0, _PTS, bink, 0, unroll=8)

            dep = ray_a + 64
            c_first = cdf_vm[pl.ds(0, _L)]
            d_first = plsc.load_gather(a_vm, [dep])
            d_last = plsc.load_gather(a_vm, [dep + (_PTS - 1)])

            def sample(j, carry):
                f_prev, cnt = carry
                cnt = cnt + hist_vm[pl.ds(j * _L, _L)]
                i = jnp.clip(cnt, 1, _PTS - 1)
                g0 = (i - 1) * _L + iota
                c0 = plsc.load_gather(cdf_vm, [g0])
                c1 = plsc.load_gather(cdf_vm, [g0 + _L])
                di = dep + (i - 1)
                d0 = plsc.load_gather(a_vm, [di])
                d1 = plsc.load_gather(a_vm, [di + 1])
                uj = lax.convert_element_type(j, jnp.float32) * (1.0 / _IMP)
                u = uj * ctot
                f = d0 + ((u - c0) / (c1 - c0)) * (d1 - d0)
                f = jnp.where(u < c_first, d_first, f)
                f = jnp.where(u >= ctot, d_last, f)
                mid = 0.5 * (f_prev + f)
                ob = ray_y + (j + 3)
                plsc.store_scatter(out_vm, [ob], o0 + mid * e0)
                plsc.store_scatter(out_vm, [ob + 64], o1 + mid * e1)
                plsc.store_scatter(out_vm, [ob + 128], o2 + mid * e2)
                return f, cnt

            cnt0 = hist_vm[pl.ds(0, _L)]
            lax.fori_loop(1, _IMP + 1, sample, (d_first, cnt0), unroll=4)
            return 0

        def chunk_body(ch, _):
            base = wid * rpw + ch * _CH
            pltpu.sync_copy(a_h.at[pl.ds(base * 128, _CH * 128)], a_vm)
            pltpu.sync_copy(b_h.at[pl.ds(base * 256, _CH * 256)], b_vm)
            pltpu.sync_copy(ori_h.at[pl.ds(base * 3, _CH * 3)], ori_vm)
            pltpu.sync_copy(dir_h.at[pl.ds(base * 3, _CH * 3)], dir_vm)
            lax.fori_loop(0, groups, group_body, 0)
            pltpu.sync_copy(out_vm, out_h.at[pl.ds(base * 256, _CH * 256)])
            return 0

        lax.fori_loop(0, nch, chunk_body, 0)

    return body(a_f, b_f, ori_f, dir_f)


def kernel(opacities, values, depths, origins, dirs):
    n = opacities.shape[0]
    a2 = (jnp.pad(opacities, ((0, 0), (0, 64)))
          + jnp.pad(depths, ((0, 0), (64, 0))))
    b2 = (jnp.pad(values[:, :, 0], ((0, 0), (0, 192)))
          + jnp.pad(values[:, :, 1], ((0, 0), (64, 128)))
          + jnp.pad(values[:, :, 2], ((0, 0), (128, 64))))
    y = _sc_call(n, a2.reshape(-1), b2.reshape(-1),
                 origins.reshape(-1), dirs.reshape(-1)).reshape(n, 256)
    accv = y[:, 0:3][:, None, :]
    acco = jnp.broadcast_to(y[:, 3:4][:, None, :], (n, 1, 3))
    coords = jnp.stack([y[:, 4:68], y[:, 68:132], y[:, 132:196]], axis=-1)
    return jnp.concatenate([accv, acco, coords], axis=1)


# speedup vs baseline: 6.5936x; 1.0870x over previous
import functools

import jax
import jax.numpy as jnp
from jax import lax
from jax.experimental import pallas as pl
from jax.experimental.pallas import tpu as pltpu
from jax.experimental.pallas import tpu_sc as plsc

_PTS = 64
_IMP = 64
_EPS = 1e-5
_CH = 64
_L = 16


def _sc_call(n, a_f, b_f):
    info = plsc.get_sparse_core_info()
    nc, ns = info.num_cores, info.num_subcores
    nw = nc * ns
    rpw = n // nw
    nch = rpw // _CH
    npair = nch // 2
    groups = _CH // _L
    asz = _CH * 128
    bsz = _CH * 256
    ysz = _CH * 256

    mesh = plsc.VectorSubcoreMesh(core_axis_name="c", subcore_axis_name="s")

    @functools.partial(
        pl.kernel,
        out_type=jax.ShapeDtypeStruct((n * 256,), jnp.float32),
        mesh=mesh,
        scratch_types=[
            pltpu.VMEM((2 * asz,), jnp.float32),
            pltpu.VMEM((2 * bsz,), jnp.float32),
            pltpu.VMEM((2 * ysz,), jnp.float32),
            pltpu.VMEM((_PTS * _L,), jnp.float32),
            pltpu.VMEM(((_IMP + 2) * _L,), jnp.int32),
            pltpu.SemaphoreType.DMA,
            pltpu.SemaphoreType.DMA,
            pltpu.SemaphoreType.DMA,
            pltpu.SemaphoreType.DMA,
            pltpu.SemaphoreType.DMA,
            pltpu.SemaphoreType.DMA,
        ],
        compiler_params=pltpu.CompilerParams(needs_layout_passes=False),
    )
    def body(a_h, b_h, out_h, a_vm, b_vm, out_vm, cdf_vm, hist_vm,
             sa0, sa1, sb0, sb1, so0, so1):
        wid = lax.axis_index("s") * nc + lax.axis_index("c")
        iota = lax.iota(jnp.int32, _L)
        zf = jnp.zeros((_L,), jnp.float32)
        onef = jnp.ones((_L,), jnp.float32)
        onei = jnp.ones((_L,), jnp.int32)
        zi = jnp.zeros((_L,), jnp.int32)
        sa = (sa0, sa1)
        sb = (sb0, sb1)
        so = (so0, so1)

        def in_copies(ch, slot):
            base = wid * rpw + ch * _CH
            ca = pltpu.make_async_copy(
                a_h.at[pl.ds(base * 128, asz)],
                a_vm.at[pl.ds(slot * asz, asz)], sa[slot])
            cb = pltpu.make_async_copy(
                b_h.at[pl.ds(base * 256, bsz)],
                b_vm.at[pl.ds(slot * bsz, bsz)], sb[slot])
            return ca, cb

        def out_copy(ch, slot):
            base = wid * rpw + ch * _CH
            return pltpu.make_async_copy(
                out_vm.at[pl.ds(slot * ysz, ysz)],
                out_h.at[pl.ds(base * 256, ysz)], so[slot])

        def compute_group(g, slot):
            aoff = slot * asz
            boff = slot * bsz
            yoff = slot * ysz
            ray = g * _L + iota
            ray_a = aoff + ray * 128
            ray_b = boff + ray * 256
            ray_y = yoff + ray * 256
            ob = ray_b + 192
            o0 = plsc.load_gather(b_vm, [ob])
            o1 = plsc.load_gather(b_vm, [ob + 1])
            o2 = plsc.load_gather(b_vm, [ob + 2])
            e0 = plsc.load_gather(b_vm, [ob + 3])
            e1 = plsc.load_gather(b_vm, [ob + 4])
            e2 = plsc.load_gather(b_vm, [ob + 5])

            def march(k, carry):
                trans, cdf, a0, a1, a2 = carry
                opk = plsc.load_gather(a_vm, [ray_a + k])
                w = opk * trans
                trans = trans * (1.0 - opk)
                cdf = cdf + (w + _EPS)
                cdf_vm[pl.ds(k * _L, _L)] = cdf
                vb = ray_b + k
                a0 = a0 + w * plsc.load_gather(b_vm, [vb])
                a1 = a1 + w * plsc.load_gather(b_vm, [vb + 64])
                a2 = a2 + w * plsc.load_gather(b_vm, [vb + 128])
                return trans, cdf, a0, a1, a2

            _, ctot, a0, a1, a2 = lax.fori_loop(
                0, _PTS, march, (onef, zf, zf, zf, zf), unroll=8)
            acc_o = jnp.clip(ctot - _PTS * _EPS, 0.0, 1.0)
            plsc.store_scatter(out_vm, [ray_y], a0)
            plsc.store_scatter(out_vm, [ray_y + 1], a1)
            plsc.store_scatter(out_vm, [ray_y + 2], a2)
            plsc.store_scatter(out_vm, [ray_y + 3], acc_o)

            def hzero(v, _):
                hist_vm[pl.ds(v * _L, _L)] = zi
                return 0

            lax.fori_loop(0, _IMP + 2, hzero, 0, unroll=8)
            scale = jnp.float32(_IMP) / ctot

            def bink(k, _):
                ck = cdf_vm[pl.ds(k * _L, _L)]
                x = ck * scale
                xi = x.astype(jnp.int32)
                xi = xi + (xi.astype(jnp.float32) < x).astype(jnp.int32)
                m = jnp.minimum(xi, _IMP + 1)
                plsc.addupdate_scatter(hist_vm, [m * _L + iota], onei)
                return 0

            lax.fori_loop(0, _PTS, bink, 0, unroll=8)

            dep = ray_a + 64
            c_first = cdf_vm[pl.ds(0, _L)]
            d_first = plsc.load_gather(a_vm, [dep])
            d_last = plsc.load_gather(a_vm, [dep + (_PTS - 1)])

            def sample(j, carry):
                f_prev, cnt = carry
                cnt = cnt + hist_vm[pl.ds(j * _L, _L)]
                i = jnp.clip(cnt, 1, _PTS - 1)
                g0 = (i - 1) * _L + iota
                c0 = plsc.load_gather(cdf_vm, [g0])
                c1 = plsc.load_gather(cdf_vm, [g0 + _L])
                di = dep + (i - 1)
                d0 = plsc.load_gather(a_vm, [di])
                d1 = plsc.load_gather(a_vm, [di + 1])
                uj = lax.convert_element_type(j, jnp.float32) * (1.0 / _IMP)
                u = uj * ctot
                f = d0 + ((u - c0) / (c1 - c0)) * (d1 - d0)
                f = jnp.where(u < c_first, d_first, f)
                f = jnp.where(u >= ctot, d_last, f)
                mid = 0.5 * (f_prev + f)
                ob = ray_y + (j + 3)
                plsc.store_scatter(out_vm, [ob], o0 + mid * e0)
                plsc.store_scatter(out_vm, [ob + 64], o1 + mid * e1)
                plsc.store_scatter(out_vm, [ob + 128], o2 + mid * e2)
                return f, cnt

            cnt0 = hist_vm[pl.ds(0, _L)]
            lax.fori_loop(1, _IMP + 1, sample, (d_first, cnt0), unroll=8)
            return 0

        def compute_chunk(slot):
            lax.fori_loop(0, groups, lambda g, _: compute_group(g, slot), 0)

        def half(p, ch, slot):
            ca, cb = in_copies(ch, slot)
            ca.wait()
            cb.wait()
            @pl.when(p > 0)
            def _():
                out_copy(ch - 2, slot).wait()

            compute_chunk(slot)
            out_copy(ch, slot).start()
            @pl.when(ch + 2 < nch)
            def _():
                na, nb = in_copies(ch + 2, slot)
                na.start()
                nb.start()

        def pair_body(p, _):
            ch0 = p * 2
            half(p, ch0, 0)
            half(p, ch0 + 1, 1)
            return 0

        pa, pb = in_copies(0, 0)
        pa.start()
        pb.start()
        qa, qb = in_copies(1, 1)
        qa.start()
        qb.start()
        lax.fori_loop(0, npair, pair_body, 0)
        out_copy(nch - 2, 0).wait()
        out_copy(nch - 1, 1).wait()

    return body(a_f, b_f)


def kernel(opacities, values, depths, origins, dirs):
    n = opacities.shape[0]
    a2 = (jnp.pad(opacities, ((0, 0), (0, 64)))
          + jnp.pad(depths, ((0, 0), (64, 0))))
    b2 = (jnp.pad(values[:, :, 0], ((0, 0), (0, 192)))
          + jnp.pad(values[:, :, 1], ((0, 0), (64, 128)))
          + jnp.pad(values[:, :, 2], ((0, 0), (128, 64)))
          + jnp.pad(origins, ((0, 0), (192, 61)))
          + jnp.pad(dirs, ((0, 0), (195, 58))))
    y = _sc_call(n, a2.reshape(-1), b2.reshape(-1)).reshape(n, 256)
    accv = y[:, 0:3][:, None, :]
    acco = jnp.broadcast_to(y[:, 3:4][:, None, :], (n, 1, 3))
    coords = jnp.stack([y[:, 4:68], y[:, 68:132], y[:, 132:196]], axis=-1)
    return jnp.concatenate([accv, acco, coords], axis=1)


# speedup vs baseline: 8.2004x; 1.2437x over previous
import functools

import jax
import jax.numpy as jnp
from jax import lax
from jax.experimental import pallas as pl
from jax.experimental.pallas import tpu as pltpu
from jax.experimental.pallas import tpu_sc as plsc

_PTS = 64
_IMP = 64
_EPS = 1e-5
_CH = 64
_L = 16


def _sc_call(n, a_f, b_f):
    info = plsc.get_sparse_core_info()
    nc, ns = info.num_cores, info.num_subcores
    nw = nc * ns
    rpw = n // nw
    nch = rpw // _CH
    npair = nch // 2
    groups = _CH // _L
    asz = _CH * 128
    bsz = _CH * 256
    ysz = _CH * 256

    mesh = plsc.VectorSubcoreMesh(core_axis_name="c", subcore_axis_name="s")

    @functools.partial(
        pl.kernel,
        out_type=jax.ShapeDtypeStruct((n * 256,), jnp.float32),
        mesh=mesh,
        scratch_types=[
            pltpu.VMEM((2 * asz,), jnp.float32),
            pltpu.VMEM((2 * bsz,), jnp.float32),
            pltpu.VMEM((2 * ysz,), jnp.float32),
            pltpu.VMEM((_PTS * _L,), jnp.float32),
            pltpu.VMEM(((_IMP + 2) * _L,), jnp.int32),
            pltpu.SemaphoreType.DMA,
            pltpu.SemaphoreType.DMA,
            pltpu.SemaphoreType.DMA,
            pltpu.SemaphoreType.DMA,
            pltpu.SemaphoreType.DMA,
            pltpu.SemaphoreType.DMA,
        ],
        compiler_params=pltpu.CompilerParams(needs_layout_passes=False),
    )
    def body(a_h, b_h, out_h, a_vm, b_vm, out_vm, cdf_vm, hist_vm,
             sa0, sa1, sb0, sb1, so0, so1):
        wid = lax.axis_index("s") * nc + lax.axis_index("c")
        iota = lax.iota(jnp.int32, _L)
        zf = jnp.zeros((_L,), jnp.float32)
        onef = jnp.ones((_L,), jnp.float32)
        onei = jnp.ones((_L,), jnp.int32)
        zi = jnp.zeros((_L,), jnp.int32)
        sa = (sa0, sa1)
        sb = (sb0, sb1)
        so = (so0, so1)

        def in_copies(ch, slot):
            base = wid * rpw + ch * _CH
            ca = pltpu.make_async_copy(
                a_h.at[pl.ds(base * 128, asz)],
                a_vm.at[pl.ds(slot * asz, asz)], sa[slot])
            cb = pltpu.make_async_copy(
                b_h.at[pl.ds(base * 256, bsz)],
                b_vm.at[pl.ds(slot * bsz, bsz)], sb[slot])
            return ca, cb

        def out_copy(ch, slot):
            base = wid * rpw + ch * _CH
            return pltpu.make_async_copy(
                out_vm.at[pl.ds(slot * ysz, ysz)],
                out_h.at[pl.ds(base * 256, ysz)], so[slot])

        def compute_group(g, slot):
            aoff = slot * asz
            boff = slot * bsz
            yoff = slot * ysz
            ray = g * _L + iota
            ray_a = aoff + ray * 128
            ray_b = boff + ray * 256
            ray_y = yoff + ray * 256
            ob = ray_b + 192
            o0 = plsc.load_gather(b_vm, [ob])
            o1 = plsc.load_gather(b_vm, [ob + 1])
            o2 = plsc.load_gather(b_vm, [ob + 2])
            e0 = plsc.load_gather(b_vm, [ob + 3])
            e1 = plsc.load_gather(b_vm, [ob + 4])
            e2 = plsc.load_gather(b_vm, [ob + 5])

            @plsc.parallel_loop(0, _PTS, unroll=8,
                               carry=(onef, zf, zf, zf, zf))
            def march(k, carry):
                trans, cdf, a0, a1, a2 = carry
                opk = plsc.load_gather(a_vm, [ray_a + k])
                w = opk * trans
                trans = trans * (1.0 - opk)
                cdf = cdf + (w + _EPS)
                cdf_vm[pl.ds(k * _L, _L)] = cdf
                vb = ray_b + k
                a0 = a0 + w * plsc.load_gather(b_vm, [vb])
                a1 = a1 + w * plsc.load_gather(b_vm, [vb + 64])
                a2 = a2 + w * plsc.load_gather(b_vm, [vb + 128])
                return trans, cdf, a0, a1, a2

            _, ctot, a0, a1, a2 = march
            acc_o = jnp.clip(ctot - _PTS * _EPS, 0.0, 1.0)
            plsc.store_scatter(out_vm, [ray_y], a0)
            plsc.store_scatter(out_vm, [ray_y + 1], a1)
            plsc.store_scatter(out_vm, [ray_y + 2], a2)
            plsc.store_scatter(out_vm, [ray_y + 3], acc_o)

            @plsc.parallel_loop(0, _IMP + 2, unroll=8)
            def hzero(v):
                hist_vm[pl.ds(v * _L, _L)] = zi

            scale = jnp.float32(_IMP) / ctot

            @plsc.parallel_loop(0, _PTS, unroll=8)
            def bink(k):
                ck = cdf_vm[pl.ds(k * _L, _L)]
                x = ck * scale
                xi = x.astype(jnp.int32)
                xi = xi + (xi.astype(jnp.float32) < x).astype(jnp.int32)
                m = jnp.minimum(xi, _IMP + 1)
                plsc.addupdate_scatter(hist_vm, [m * _L + iota], onei)

            dep = ray_a + 64
            c_first = cdf_vm[pl.ds(0, _L)]
            d_first = plsc.load_gather(a_vm, [dep])
            d_last = plsc.load_gather(a_vm, [dep + (_PTS - 1)])

            @plsc.parallel_loop(1, _IMP + 1, unroll=8,
                               carry=(d_first, hist_vm[pl.ds(0, _L)]))
            def sample(j, carry):
                f_prev, cnt = carry
                cnt = cnt + hist_vm[pl.ds(j * _L, _L)]
                i = jnp.clip(cnt, 1, _PTS - 1)
                g0 = (i - 1) * _L + iota
                c0 = plsc.load_gather(cdf_vm, [g0])
                c1 = plsc.load_gather(cdf_vm, [g0 + _L])
                di = dep + (i - 1)
                d0 = plsc.load_gather(a_vm, [di])
                d1 = plsc.load_gather(a_vm, [di + 1])
                uj = lax.convert_element_type(j, jnp.float32) * (1.0 / _IMP)
                u = uj * ctot
                f = d0 + ((u - c0) / (c1 - c0)) * (d1 - d0)
                f = jnp.where(u < c_first, d_first, f)
                f = jnp.where(u >= ctot, d_last, f)
                mid = 0.5 * (f_prev + f)
                ob = ray_y + (j + 3)
                plsc.store_scatter(out_vm, [ob], o0 + mid * e0)
                plsc.store_scatter(out_vm, [ob + 64], o1 + mid * e1)
                plsc.store_scatter(out_vm, [ob + 128], o2 + mid * e2)
                return f, cnt
            del sample
            return 0

        def compute_chunk(slot):
            lax.fori_loop(0, groups, lambda g, _: compute_group(g, slot), 0)

        def half(p, ch, slot):
            ca, cb = in_copies(ch, slot)
            ca.wait()
            cb.wait()
            @pl.when(p > 0)
            def _():
                out_copy(ch - 2, slot).wait()

            compute_chunk(slot)
            out_copy(ch, slot).start()
            @pl.when(ch + 2 < nch)
            def _():
                na, nb = in_copies(ch + 2, slot)
                na.start()
                nb.start()

        def pair_body(p, _):
            ch0 = p * 2
            half(p, ch0, 0)
            half(p, ch0 + 1, 1)
            return 0

        pa, pb = in_copies(0, 0)
        pa.start()
        pb.start()
        qa, qb = in_copies(1, 1)
        qa.start()
        qb.start()
        lax.fori_loop(0, npair, pair_body, 0)
        out_copy(nch - 2, 0).wait()
        out_copy(nch - 1, 1).wait()

    return body(a_f, b_f)


def kernel(opacities, values, depths, origins, dirs):
    n = opacities.shape[0]
    a2 = (jnp.pad(opacities, ((0, 0), (0, 64)))
          + jnp.pad(depths, ((0, 0), (64, 0))))
    b2 = (jnp.pad(values[:, :, 0], ((0, 0), (0, 192)))
          + jnp.pad(values[:, :, 1], ((0, 0), (64, 128)))
          + jnp.pad(values[:, :, 2], ((0, 0), (128, 64)))
          + jnp.pad(origins, ((0, 0), (192, 61)))
          + jnp.pad(dirs, ((0, 0), (195, 58))))
    y = _sc_call(n, a2.reshape(-1), b2.reshape(-1)).reshape(n, 256)
    accv = y[:, 0:3][:, None, :]
    acco = jnp.broadcast_to(y[:, 3:4][:, None, :], (n, 1, 3))
    coords = jnp.stack([y[:, 4:68], y[:, 68:132], y[:, 132:196]], axis=-1)
    return jnp.concatenate([accv, acco, coords], axis=1)


# speedup vs baseline: 8.5616x; 1.0440x over previous
import functools

import jax
import jax.numpy as jnp
from jax import lax
from jax.experimental import pallas as pl
from jax.experimental.pallas import tpu as pltpu
from jax.experimental.pallas import tpu_sc as plsc

_PTS = 64
_IMP = 64
_EPS = 1e-5
_CH = 64
_L = 16


def _sc_call(n, a_f, b1_f, b2_f):
    info = plsc.get_sparse_core_info()
    nc, ns = info.num_cores, info.num_subcores
    nw = nc * ns
    rpw = n // nw
    nch = rpw // _CH
    npair = nch // 2
    groups = _CH // _L
    csz = _CH * 128
    cdfg = _PTS * _L
    histg = (_IMP + 2) * _L

    mesh = plsc.VectorSubcoreMesh(core_axis_name="c", subcore_axis_name="s")

    @functools.partial(
        pl.kernel,
        out_type=[jax.ShapeDtypeStruct((n * 128,), jnp.float32),
                  jax.ShapeDtypeStruct((n * 128,), jnp.float32)],
        mesh=mesh,
        scratch_types=[
            pltpu.VMEM((2 * csz,), jnp.float32),
            pltpu.VMEM((2 * csz,), jnp.float32),
            pltpu.VMEM((2 * csz,), jnp.float32),
            pltpu.VMEM((2 * csz,), jnp.float32),
            pltpu.VMEM((2 * csz,), jnp.float32),
            pltpu.VMEM((4 * cdfg,), jnp.float32),
            pltpu.VMEM((4 * histg,), jnp.int32),
            pltpu.SemaphoreType.DMA,
            pltpu.SemaphoreType.DMA,
            pltpu.SemaphoreType.DMA,
            pltpu.SemaphoreType.DMA,
            pltpu.SemaphoreType.DMA,
            pltpu.SemaphoreType.DMA,
            pltpu.SemaphoreType.DMA,
            pltpu.SemaphoreType.DMA,
        ],
        compiler_params=pltpu.CompilerParams(needs_layout_passes=False),
    )
    def body(a_h, b1_h, b2_h, y1_h, y2_h,
             a_vm, b1_vm, b2_vm, y1_vm, y2_vm, cdf_vm, hist_vm,
             sa0, sa1, sb0, sb1, sc0, sc1, so0, so1):
        wid = lax.axis_index("s") * nc + lax.axis_index("c")
        iota = lax.iota(jnp.int32, _L)
        zf = jnp.zeros((_L,), jnp.float32)
        onef = jnp.ones((_L,), jnp.float32)
        onei = jnp.ones((_L,), jnp.int32)
        zi = jnp.zeros((_L,), jnp.int32)
        sa = (sa0, sa1)
        sb = (sb0, sb1)
        sc = (sc0, sc1)
        so = (so0, so1)

        def in_copies(ch, slot):
            off = wid * rpw * 128 + ch * csz
            hb = pl.ds(off, csz)
            vm = pl.ds(slot * csz, csz)
            return (
                pltpu.make_async_copy(a_h.at[hb], a_vm.at[vm], sa[slot]),
                pltpu.make_async_copy(b1_h.at[hb], b1_vm.at[vm], sb[slot]),
                pltpu.make_async_copy(b2_h.at[hb], b2_vm.at[vm], sc[slot]),
            )

        def out_copies(ch, slot):
            off = wid * rpw * 128 + ch * csz
            hb = pl.ds(off, csz)
            vm = pl.ds(slot * csz, csz)
            return (
                pltpu.make_async_copy(y1_vm.at[vm], y1_h.at[hb], so[slot]),
                pltpu.make_async_copy(y2_vm.at[vm], y2_h.at[hb], so[slot]),
            )

        def compute_group(g, slot):
            voff = slot * csz
            ray = g * _L + iota
            ray_p = voff + ray * 128
            coff = g * cdfg
            hoff = g * histg
            ob = ray_p + 64
            o0 = plsc.load_gather(b2_vm, [ob])
            o1 = plsc.load_gather(b2_vm, [ob + 1])
            o2 = plsc.load_gather(b2_vm, [ob + 2])
            e0 = plsc.load_gather(b2_vm, [ob + 3])
            e1 = plsc.load_gather(b2_vm, [ob + 4])
            e2 = plsc.load_gather(b2_vm, [ob + 5])

            @plsc.parallel_loop(0, _PTS, unroll=8,
                               carry=(onef, zf, zf, zf, zf))
            def march(k, carry):
                trans, cdf, a0, a1, a2 = carry
                opk = plsc.load_gather(a_vm, [ray_p + k])
                w = opk * trans
                trans = trans * (1.0 - opk)
                cdf = cdf + (w + _EPS)
                cdf_vm[pl.ds(coff + k * _L, _L)] = cdf
                vb = ray_p + k
                a0 = a0 + w * plsc.load_gather(b1_vm, [vb])
                a1 = a1 + w * plsc.load_gather(b1_vm, [vb + 64])
                a2 = a2 + w * plsc.load_gather(b2_vm, [vb])
                return trans, cdf, a0, a1, a2

            _, ctot, a0, a1, a2 = march
            acc_o = jnp.clip(ctot - _PTS * _EPS, 0.0, 1.0)
            ya = ray_p + 64
            plsc.store_scatter(y2_vm, [ya], a0)
            plsc.store_scatter(y2_vm, [ya + 1], a1)
            plsc.store_scatter(y2_vm, [ya + 2], a2)
            plsc.store_scatter(y2_vm, [ya + 3], acc_o)

            @plsc.parallel_loop(0, _IMP + 2, unroll=8)
            def hzero(v):
                hist_vm[pl.ds(hoff + v * _L, _L)] = zi

            scale = jnp.float32(_IMP) / ctot

            @plsc.parallel_loop(0, _PTS, unroll=8)
            def bink(k):
                ck = cdf_vm[pl.ds(coff + k * _L, _L)]
                x = ck * scale
                xi = x.astype(jnp.int32)
                xi = xi + (xi.astype(jnp.float32) < x).astype(jnp.int32)
                m = jnp.minimum(xi, _IMP + 1)
                plsc.addupdate_scatter(hist_vm, [hoff + m * _L + iota], onei)

            dep = ray_p + 64
            c_first = cdf_vm[pl.ds(coff, _L)]
            d_first = plsc.load_gather(a_vm, [dep])
            d_last = plsc.load_gather(a_vm, [dep + (_PTS - 1)])

            @plsc.parallel_loop(1, _IMP + 1, unroll=8,
                               carry=(d_first, hist_vm[pl.ds(hoff, _L)]))
            def sample(j, carry):
                f_prev, cnt = carry
                cnt = cnt + hist_vm[pl.ds(hoff + j * _L, _L)]
                i = jnp.clip(cnt, 1, _PTS - 1)
                g0 = coff + (i - 1) * _L + iota
                c0 = plsc.load_gather(cdf_vm, [g0])
                c1 = plsc.load_gather(cdf_vm, [g0 + _L])
                di = dep + (i - 1)
                d0 = plsc.load_gather(a_vm, [di])
                d1 = plsc.load_gather(a_vm, [di + 1])
                uj = lax.convert_element_type(j, jnp.float32) * (1.0 / _IMP)
                u = uj * ctot
                f = d0 + ((u - c0) / (c1 - c0)) * (d1 - d0)
                f = jnp.where(u < c_first, d_first, f)
                f = jnp.where(u >= ctot, d_last, f)
                mid = 0.5 * (f_prev + f)
                yb = ray_p + (j - 1)
                plsc.store_scatter(y1_vm, [yb], o0 + mid * e0)
                plsc.store_scatter(y1_vm, [yb + 64], o1 + mid * e1)
                plsc.store_scatter(y2_vm, [yb], o2 + mid * e2)
                return f, cnt

            del sample

        def compute_chunk(slot):
            @plsc.parallel_loop(0, groups, unroll=2)
            def grp(g):
                compute_group(g, slot)

        def half(p, ch, slot):
            ca, cb, cc = in_copies(ch, slot)
            ca.wait()
            cb.wait()
            cc.wait()
            @pl.when(p > 0)
            def _():
                oa, ob_ = out_copies(ch - 2, slot)
                oa.wait()
                ob_.wait()

            compute_chunk(slot)
            oa, ob_ = out_copies(ch, slot)
            oa.start()
            ob_.start()
            @pl.when(ch + 2 < nch)
            def _():
                na, nb, ncp = in_copies(ch + 2, slot)
                na.start()
                nb.start()
                ncp.start()

        def pair_body(p, _):
            ch0 = p * 2
            half(p, ch0, 0)
            half(p, ch0 + 1, 1)
            return 0

        pa, pb, pc = in_copies(0, 0)
        pa.start()
        pb.start()
        pc.start()
        qa, qb, qc = in_copies(1, 1)
        qa.start()
        qb.start()
        qc.start()
        lax.fori_loop(0, npair, pair_body, 0)
        fa, fb = out_copies(nch - 2, 0)
        fa.wait()
        fb.wait()
        ga, gb = out_copies(nch - 1, 1)
        ga.wait()
        gb.wait()

    return body(a_f, b1_f, b2_f)


def kernel(opacities, values, depths, origins, dirs):
    n = opacities.shape[0]
    a2 = (jnp.pad(opacities, ((0, 0), (0, 64)))
          + jnp.pad(depths, ((0, 0), (64, 0))))
    b1 = (jnp.pad(values[:, :, 0], ((0, 0), (0, 64)))
          + jnp.pad(values[:, :, 1], ((0, 0), (64, 0))))
    b2 = (jnp.pad(values[:, :, 2], ((0, 0), (0, 64)))
          + jnp.pad(origins, ((0, 0), (64, 61)))
          + jnp.pad(dirs, ((0, 0), (67, 58))))
    y1, y2 = _sc_call(n, a2.reshape(-1), b1.reshape(-1), b2.reshape(-1))
    y1 = y1.reshape(n, 128)
    y2 = y2.reshape(n, 128)
    accv = y2[:, 64:67][:, None, :]
    acco = jnp.broadcast_to(y2[:, 67:68][:, None, :], (n, 1, 3))
    coords = jnp.stack([y1[:, 0:64], y1[:, 64:128], y2[:, 0:64]], axis=-1)
    return jnp.concatenate([accv, acco, coords], axis=1)
